# baseline (speedup 1.0000x reference)
import functools

import jax
import jax.numpy as jnp
from jax import lax
from jax.experimental import pallas as pl
from jax.experimental.pallas import tpu as pltpu

B, C, H, W = 4, 192, 224, 224
HEADS = 4
N = B * 4
HW = H * W
ROWS = B * C
RB = 16


def _means_body(x_ref, r_ref, d_ref, o_ref):
    inv = 1.0 / HW
    o_ref[0, :, :] = jnp.sum(x_ref[...], axis=1, keepdims=True) * inv
    o_ref[1, :, :] = jnp.sum(r_ref[...], axis=1, keepdims=True) * inv
    o_ref[2, :, :] = jnp.sum(d_ref[...], axis=1, keepdims=True) * inv


def _means(x2d, r2d, d2d):
    grid = ROWS // RB
    return pl.pallas_call(
        _means_body,
        grid=(grid,),
        in_specs=[pl.BlockSpec((RB, HW), lambda i: (i, 0))] * 3,
        out_specs=pl.BlockSpec((3, RB, 1), lambda i: (0, i, 0)),
        out_shape=jax.ShapeDtypeStruct((3, ROWS, 1), jnp.float32),
    )(x2d, r2d, d2d)


def _adj_mask():
    r = lax.broadcasted_iota(jnp.int32, (N, N), 0)
    c = lax.broadcasted_iota(jnp.int32, (N, N), 1)
    same = (r // 4) == (c // 4)
    nr, nc = r % 4, c % 4
    adj = (nc >= 1) | ((nr == 0) & (nc == 0))
    return same & adj


def _gat_layer(g, Wm, a_s, a_d, bb, mask, maskf):
    h = jnp.dot(g, Wm, preferred_element_type=jnp.float32)
    acc = jnp.zeros((N, C), jnp.float32)
    for hd in range(HEADS):
        hh = h[:, hd * C:(hd + 1) * C]
        a_s_h = a_s[hd:hd + 1, :]
        a_d_h = a_d[hd:hd + 1, :]
        al_s = lax.dot_general(a_s_h, hh, (((1,), (1,)), ((), ())),
                               preferred_element_type=jnp.float32)
        al_d = lax.dot_general(hh, a_d_h, (((1,), (1,)), ((), ())),
                               preferred_element_type=jnp.float32)
        e = al_d + al_s
        e = jnp.where(e > 0, e, 0.2 * e)
        e = jnp.where(mask, e, -1e30)
        m = jnp.max(e, axis=1, keepdims=True)
        ex = jnp.exp(e - m) * maskf
        ssum = jnp.sum(ex, axis=1, keepdims=True) + 1e-16
        alpha = ex / ssum
        acc = acc + jnp.dot(alpha, hh, preferred_element_type=jnp.float32)
    return acc * (1.0 / HEADS) + bb


def _ln(x, g, b):
    mu = jnp.mean(x, axis=-1, keepdims=True)
    xc = x - mu
    var = jnp.mean(xc * xc, axis=-1, keepdims=True)
    return xc * lax.rsqrt(var + 1e-5) * g + b


def _gnn_body(tok_ref, mean_ref, W0_ref, as0_ref, ad0_ref, b0_ref, g0_ref,
              be0_ref, W1_ref, as1_ref, ad1_ref, b1_ref, g1_ref, be1_ref,
              o_ref):
    t = jnp.broadcast_to(tok_ref[...], (B, C))
    fu = mean_ref[0]
    x1 = mean_ref[1]
    x2 = mean_ref[2]
    feats = jnp.stack([t, fu, x1, x2], axis=1).reshape(N, C)

    mask = _adj_mask()
    maskf = mask.astype(jnp.float32)

    g = feats
    for (Wr, ar_s, ar_d, br, lgr, lbr) in (
            (W0_ref, as0_ref, ad0_ref, b0_ref, g0_ref, be0_ref),
            (W1_ref, as1_ref, ad1_ref, b1_ref, g1_ref, be1_ref)):
        g = _gat_layer(g, Wr[...], ar_s[...], ar_d[...], br[...], mask,
                       maskf) + g
        g = _ln(g, lgr[...], lbr[...])
        g = jnp.maximum(g, 0.0)

    rr = lax.broadcasted_iota(jnp.int32, (B, N), 0)
    cc = lax.broadcasted_iota(jnp.int32, (B, N), 1)
    sel = (cc == rr * 4).astype(jnp.float32)
    gtok = jnp.dot(sel, g, preferred_element_type=jnp.float32)
    o_ref[...] = 1.0 + jax.nn.sigmoid(gtok)


def _gnn(tok, means, W0, as0, ad0, b0, g0, be0, W1, as1, ad1, b1, g1, be1):
    full = lambda s: pl.BlockSpec(s, lambda: (0,) * len(s))
    return pl.pallas_call(
        _gnn_body,
        in_specs=[full((1, C)), full((3, B, C)), full((C, HEADS * C)),
                  full((HEADS, C)), full((HEADS, C)), full((1, C)),
                  full((1, C)), full((1, C)), full((C, HEADS * C)),
                  full((HEADS, C)), full((HEADS, C)), full((1, C)),
                  full((1, C)), full((1, C))],
        out_specs=full((B, C)),
        out_shape=jax.ShapeDtypeStruct((B, C), jnp.float32),
    )(tok, means, W0, as0, ad0, b0, g0, be0, W1, as1, ad1, b1, g1, be1)


def _scale_body(x_ref, s_ref, o_ref):
    o_ref[...] = x_ref[...] * s_ref[...]


def _scale(x2d, s):
    grid = ROWS // RB
    return pl.pallas_call(
        _scale_body,
        grid=(grid,),
        in_specs=[pl.BlockSpec((RB, HW), lambda i: (i, 0)),
                  pl.BlockSpec((RB, 1), lambda i: (i, 0))],
        out_specs=pl.BlockSpec((RB, HW), lambda i: (i, 0)),
        out_shape=jax.ShapeDtypeStruct((ROWS, HW), jnp.float32),
    )(x2d, s)


def kernel(x_ful, rgb, dep, tok, W0, a_src0, a_dst0, b0, g0, be0,
           W1, a_src1, a_dst1, b1, g1, be1):
    x2d = x_ful.reshape(ROWS, HW)
    r2d = rgb.reshape(ROWS, HW)
    d2d = dep.reshape(ROWS, HW)

    means = _means(x2d, r2d, d2d).reshape(3, B, C)

    scale = _gnn(tok, means,
                 W0, a_src0.reshape(HEADS, C), a_dst0.reshape(HEADS, C),
                 b0.reshape(1, C), g0.reshape(1, C), be0.reshape(1, C),
                 W1, a_src1.reshape(HEADS, C), a_dst1.reshape(HEADS, C),
                 b1.reshape(1, C), g1.reshape(1, C), be1.reshape(1, C))

    out = _scale(x2d, scale.reshape(ROWS, 1))
    return out.reshape(B, C, H, W)


# speedup vs baseline: 3.3234x; 3.3234x over previous
import functools

import jax
import jax.numpy as jnp
from jax import lax
from jax.experimental import pallas as pl
from jax.experimental.pallas import tpu as pltpu

B, C, H, W = 4, 192, 224, 224
HEADS = 4
N = B * 4
ROWS = B * C
CB = 16


def _means_body(x_ref, r_ref, d_ref, o_ref):
    inv = 1.0 / (H * W)
    o_ref[0, 0:1, :] = jnp.sum(x_ref[...], axis=(2, 3)) * inv
    o_ref[0, 1:2, :] = jnp.sum(r_ref[...], axis=(2, 3)) * inv
    o_ref[0, 2:3, :] = jnp.sum(d_ref[...], axis=(2, 3)) * inv


def _means(x, r, d):
    grid = ROWS // CB
    nc = C // CB
    bs = pl.BlockSpec((1, CB, H, W), lambda i: (i // nc, i % nc, 0, 0))
    out = pl.pallas_call(
        _means_body,
        grid=(grid,),
        in_specs=[bs] * 3,
        out_specs=pl.BlockSpec((1, 3, CB), lambda i: (i, 0, 0)),
        out_shape=jax.ShapeDtypeStruct((grid, 3, CB), jnp.float32),
    )(x, r, d)
    return out.transpose(1, 0, 2).reshape(3, B, C)


def _adj_mask():
    r = lax.broadcasted_iota(jnp.int32, (N, N), 0)
    c = lax.broadcasted_iota(jnp.int32, (N, N), 1)
    same = (r // 4) == (c // 4)
    nr, nc = r % 4, c % 4
    adj = (nc >= 1) | ((nr == 0) & (nc == 0))
    return same & adj


def _gat_layer(g, Wm, a_s, a_d, bb, mask, maskf):
    h = jnp.dot(g, Wm, preferred_element_type=jnp.float32)
    acc = jnp.zeros((N, C), jnp.float32)
    for hd in range(HEADS):
        hh = h[:, hd * C:(hd + 1) * C]
        a_s_h = a_s[hd:hd + 1, :]
        a_d_h = a_d[hd:hd + 1, :]
        al_s = lax.dot_general(a_s_h, hh, (((1,), (1,)), ((), ())),
                               preferred_element_type=jnp.float32)
        al_d = lax.dot_general(hh, a_d_h, (((1,), (1,)), ((), ())),
                               preferred_element_type=jnp.float32)
        e = al_d + al_s
        e = jnp.where(e > 0, e, 0.2 * e)
        e = jnp.where(mask, e, -1e30)
        m = jnp.max(e, axis=1, keepdims=True)
        ex = jnp.exp(e - m) * maskf
        ssum = jnp.sum(ex, axis=1, keepdims=True) + 1e-16
        alpha = ex / ssum
        acc = acc + jnp.dot(alpha, hh, preferred_element_type=jnp.float32)
    return acc * (1.0 / HEADS) + bb


def _ln(x, g, b):
    mu = jnp.mean(x, axis=-1, keepdims=True)
    xc = x - mu
    var = jnp.mean(xc * xc, axis=-1, keepdims=True)
    return xc * lax.rsqrt(var + 1e-5) * g + b


def _gnn_body(tok_ref, mean_ref, W0_ref, as0_ref, ad0_ref, b0_ref, g0_ref,
              be0_ref, W1_ref, as1_ref, ad1_ref, b1_ref, g1_ref, be1_ref,
              o_ref):
    t = jnp.broadcast_to(tok_ref[...], (B, C))
    fu = mean_ref[0]
    x1 = mean_ref[1]
    x2 = mean_ref[2]
    feats = jnp.stack([t, fu, x1, x2], axis=1).reshape(N, C)

    mask = _adj_mask()
    maskf = mask.astype(jnp.float32)

    g = feats
    for (Wr, ar_s, ar_d, br, lgr, lbr) in (
            (W0_ref, as0_ref, ad0_ref, b0_ref, g0_ref, be0_ref),
            (W1_ref, as1_ref, ad1_ref, b1_ref, g1_ref, be1_ref)):
        g = _gat_layer(g, Wr[...], ar_s[...], ar_d[...], br[...], mask,
                       maskf) + g
        g = _ln(g, lgr[...], lbr[...])
        g = jnp.maximum(g, 0.0)

    rr = lax.broadcasted_iota(jnp.int32, (B, N), 0)
    cc = lax.broadcasted_iota(jnp.int32, (B, N), 1)
    sel = (cc == rr * 4).astype(jnp.float32)
    gtok = jnp.dot(sel, g, preferred_element_type=jnp.float32)
    o_ref[...] = 1.0 + jax.nn.sigmoid(gtok)


def _gnn(tok, means, W0, as0, ad0, b0, g0, be0, W1, as1, ad1, b1, g1, be1):
    full = lambda s: pl.BlockSpec(s, lambda: (0,) * len(s))
    return pl.pallas_call(
        _gnn_body,
        in_specs=[full((1, C)), full((3, B, C)), full((C, HEADS * C)),
                  full((HEADS, C)), full((HEADS, C)), full((1, C)),
                  full((1, C)), full((1, C)), full((C, HEADS * C)),
                  full((HEADS, C)), full((HEADS, C)), full((1, C)),
                  full((1, C)), full((1, C))],
        out_specs=full((B, C)),
        out_shape=jax.ShapeDtypeStruct((B, C), jnp.float32),
    )(tok, means, W0, as0, ad0, b0, g0, be0, W1, as1, ad1, b1, g1, be1)


def _scale_body(x_ref, s_ref, o_ref):
    i = pl.program_id(0)
    for k in range(CB):
        o_ref[0, k] = x_ref[0, k] * s_ref[i * CB + k]


def _scale(x, s):
    grid = ROWS // CB
    nc = C // CB
    bs = pl.BlockSpec((1, CB, H, W), lambda i: (i // nc, i % nc, 0, 0))
    return pl.pallas_call(
        _scale_body,
        grid=(grid,),
        in_specs=[bs, pl.BlockSpec(memory_space=pltpu.SMEM)],
        out_specs=bs,
        out_shape=jax.ShapeDtypeStruct((B, C, H, W), jnp.float32),
    )(x, s)


def kernel(x_ful, rgb, dep, tok, W0, a_src0, a_dst0, b0, g0, be0,
           W1, a_src1, a_dst1, b1, g1, be1):
    means = _means(x_ful, rgb, dep)

    scale = _gnn(tok, means,
                 W0, a_src0.reshape(HEADS, C), a_dst0.reshape(HEADS, C),
                 b0.reshape(1, C), g0.reshape(1, C), be0.reshape(1, C),
                 W1, a_src1.reshape(HEADS, C), a_dst1.reshape(HEADS, C),
                 b1.reshape(1, C), g1.reshape(1, C), be1.reshape(1, C))

    return _scale(x_ful, scale.reshape(ROWS))


# speedup vs baseline: 3.3461x; 1.0068x over previous
import functools

import jax
import jax.numpy as jnp
from jax import lax
from jax.experimental import pallas as pl
from jax.experimental.pallas import tpu as pltpu

B, C, H, W = 4, 192, 224, 224
HEADS = 4
N = B * 4
ROWS = B * C
CB = 32


def _means_body(x_ref, r_ref, d_ref, o_ref):
    inv = 1.0 / (H * W)
    o_ref[0, 0:1, :] = jnp.sum(x_ref[...], axis=(2, 3)) * inv
    o_ref[0, 1:2, :] = jnp.sum(r_ref[...], axis=(2, 3)) * inv
    o_ref[0, 2:3, :] = jnp.sum(d_ref[...], axis=(2, 3)) * inv


def _means(x, r, d):
    grid = ROWS // CB
    nc = C // CB
    bs = pl.BlockSpec((1, CB, H, W), lambda i: (i // nc, i % nc, 0, 0))
    out = pl.pallas_call(
        _means_body,
        grid=(grid,),
        in_specs=[bs] * 3,
        out_specs=pl.BlockSpec((1, 3, CB), lambda i: (i, 0, 0)),
        out_shape=jax.ShapeDtypeStruct((grid, 3, CB), jnp.float32),
    )(x, r, d)
    return out.transpose(1, 0, 2).reshape(3, B, C)


def _adj_mask():
    r = lax.broadcasted_iota(jnp.int32, (N, N), 0)
    c = lax.broadcasted_iota(jnp.int32, (N, N), 1)
    same = (r // 4) == (c // 4)
    nr, nc = r % 4, c % 4
    adj = (nc >= 1) | ((nr == 0) & (nc == 0))
    return same & adj


def _gat_layer(g, Wm, a_s, a_d, bb, mask, maskf):
    h = jnp.dot(g, Wm, preferred_element_type=jnp.float32)
    acc = jnp.zeros((N, C), jnp.float32)
    for hd in range(HEADS):
        hh = h[:, hd * C:(hd + 1) * C]
        a_s_h = a_s[hd:hd + 1, :]
        a_d_h = a_d[hd:hd + 1, :]
        al_s = lax.dot_general(a_s_h, hh, (((1,), (1,)), ((), ())),
                               preferred_element_type=jnp.float32)
        al_d = lax.dot_general(hh, a_d_h, (((1,), (1,)), ((), ())),
                               preferred_element_type=jnp.float32)
        e = al_d + al_s
        e = jnp.where(e > 0, e, 0.2 * e)
        e = jnp.where(mask, e, -1e30)
        m = jnp.max(e, axis=1, keepdims=True)
        ex = jnp.exp(e - m) * maskf
        ssum = jnp.sum(ex, axis=1, keepdims=True) + 1e-16
        alpha = ex / ssum
        acc = acc + jnp.dot(alpha, hh, preferred_element_type=jnp.float32)
    return acc * (1.0 / HEADS) + bb


def _ln(x, g, b):
    mu = jnp.mean(x, axis=-1, keepdims=True)
    xc = x - mu
    var = jnp.mean(xc * xc, axis=-1, keepdims=True)
    return xc * lax.rsqrt(var + 1e-5) * g + b


def _gnn_body(tok_ref, mean_ref, W0_ref, as0_ref, ad0_ref, b0_ref, g0_ref,
              be0_ref, W1_ref, as1_ref, ad1_ref, b1_ref, g1_ref, be1_ref,
              o_ref):
    t = jnp.broadcast_to(tok_ref[...], (B, C))
    fu = mean_ref[0]
    x1 = mean_ref[1]
    x2 = mean_ref[2]
    feats = jnp.stack([t, fu, x1, x2], axis=1).reshape(N, C)

    mask = _adj_mask()
    maskf = mask.astype(jnp.float32)

    g = feats
    for (Wr, ar_s, ar_d, br, lgr, lbr) in (
            (W0_ref, as0_ref, ad0_ref, b0_ref, g0_ref, be0_ref),
            (W1_ref, as1_ref, ad1_ref, b1_ref, g1_ref, be1_ref)):
        g = _gat_layer(g, Wr[...], ar_s[...], ar_d[...], br[...], mask,
                       maskf) + g
        g = _ln(g, lgr[...], lbr[...])
        g = jnp.maximum(g, 0.0)

    rr = lax.broadcasted_iota(jnp.int32, (B, N), 0)
    cc = lax.broadcasted_iota(jnp.int32, (B, N), 1)
    sel = (cc == rr * 4).astype(jnp.float32)
    gtok = jnp.dot(sel, g, preferred_element_type=jnp.float32)
    o_ref[...] = 1.0 + jax.nn.sigmoid(gtok)


def _gnn(tok, means, W0, as0, ad0, b0, g0, be0, W1, as1, ad1, b1, g1, be1):
    full = lambda s: pl.BlockSpec(s, lambda: (0,) * len(s))
    return pl.pallas_call(
        _gnn_body,
        in_specs=[full((1, C)), full((3, B, C)), full((C, HEADS * C)),
                  full((HEADS, C)), full((HEADS, C)), full((1, C)),
                  full((1, C)), full((1, C)), full((C, HEADS * C)),
                  full((HEADS, C)), full((HEADS, C)), full((1, C)),
                  full((1, C)), full((1, C))],
        out_specs=full((B, C)),
        out_shape=jax.ShapeDtypeStruct((B, C), jnp.float32),
    )(tok, means, W0, as0, ad0, b0, g0, be0, W1, as1, ad1, b1, g1, be1)


def _scale_body(x_ref, s_ref, o_ref):
    i = pl.program_id(0)
    for k in range(CB):
        o_ref[0, k] = x_ref[0, k] * s_ref[i * CB + k]


def _scale(x, s):
    grid = ROWS // CB
    nc = C // CB
    bs = pl.BlockSpec((1, CB, H, W), lambda i: (i // nc, i % nc, 0, 0))
    return pl.pallas_call(
        _scale_body,
        grid=(grid,),
        in_specs=[bs, pl.BlockSpec(memory_space=pltpu.SMEM)],
        out_specs=bs,
        out_shape=jax.ShapeDtypeStruct((B, C, H, W), jnp.float32),
    )(x, s)


def kernel(x_ful, rgb, dep, tok, W0, a_src0, a_dst0, b0, g0, be0,
           W1, a_src1, a_dst1, b1, g1, be1):
    means = _means(x_ful, rgb, dep)

    scale = _gnn(tok, means,
                 W0, a_src0.reshape(HEADS, C), a_dst0.reshape(HEADS, C),
                 b0.reshape(1, C), g0.reshape(1, C), be0.reshape(1, C),
                 W1, a_src1.reshape(HEADS, C), a_dst1.reshape(HEADS, C),
                 b1.reshape(1, C), g1.reshape(1, C), be1.reshape(1, C))

    return _scale(x_ful, scale.reshape(ROWS))
